# baseline (speedup 1.0000x reference)
import functools

import jax
import jax.numpy as jnp
from jax.experimental import pallas as pl
from jax.experimental.pallas import tpu as pltpu

_EPS = 0.3 ** 2
_MAX_ITER = 100
_BLOCK_N = 32
_LOG2E = 1.4426950408889634


def _prep(samples_ref, codebook_ref):
    B, S, d = samples_ref.shape
    K = codebook_ref.shape[0]
    scale = jnp.float32(_LOG2E / _EPS)
    x = samples_ref[...]
    y = codebook_ref[...]
    x2 = jnp.sum(x * x, axis=-1)
    y2 = jnp.sum(y * y, axis=-1)
    xy = jax.lax.dot_general(
        x.reshape(B * S, d), y,
        dimension_numbers=(((1,), (1,)), ((), ())),
        preferred_element_type=jnp.float32,
    )
    D0 = ((xy - 0.5 * x2.reshape(B * S, 1)) - 0.5 * y2.reshape(1, K)) * scale
    return D0.reshape(B, S, K)


def _fast_kernel(samples_ref, codebook_ref, out_ref, flag_ref, D_ref):
    B, S, _ = samples_ref.shape
    K = codebook_ref.shape[0]
    l2_a = -jnp.log2(jnp.float32(S))
    l2_b = -jnp.log2(jnp.float32(K))
    lo = jnp.float32(2.0 ** -80)
    hi = jnp.float32(2.0 ** 80)

    D0 = _prep(samples_ref, codebook_ref)

    m = jnp.max(D0, axis=2)
    s = jnp.sum(jnp.exp2(D0 - m[:, :, None]), axis=2)
    dfe = -(m + jnp.log2(s) + l2_b)
    Dn = D0 + dfe[:, :, None]
    m2 = jnp.max(Dn, axis=1)
    s2 = jnp.sum(jnp.exp2(Dn - m2[:, None, :]), axis=1)
    dge = -(m2 + jnp.log2(s2) + l2_a)
    D_ref[...] = Dn

    def body(_, carry):
        dge, bad = carry
        Dn = D_ref[...] + dge[:, None, :]
        s = jnp.sum(jnp.exp2(Dn), axis=2)
        bad = bad | jnp.any((s <= lo) | (s >= hi) | jnp.isnan(s))
        dfe = -(jnp.log2(s) + l2_b)
        D_ref[...] = Dn + dfe[:, :, None]
        s2 = jnp.sum(jnp.exp2(D_ref[...]), axis=1)
        bad = bad | jnp.any((s2 <= lo) | (s2 >= hi) | jnp.isnan(s2))
        dge = -(jnp.log2(s2) + l2_a)
        return dge, bad

    dge, bad = jax.lax.fori_loop(
        0, _MAX_ITER - 1, body, (dge, jnp.bool_(False)), unroll=33)

    h = jnp.sum(jnp.exp2(D_ref[...] + dge[:, None, :]), axis=1)
    out_ref[...] = h / jnp.sum(h, axis=1, keepdims=True)
    flag_ref[...] = jnp.full((B, K), jnp.where(bad, 1.0, 0.0), jnp.float32)


def _exact_kernel(samples_ref, codebook_ref, out_ref):
    B, S, _ = samples_ref.shape
    K = codebook_ref.shape[0]
    l2_a = -jnp.log2(jnp.float32(S))
    l2_b = -jnp.log2(jnp.float32(K))

    Ce = -_prep(samples_ref, codebook_ref)

    def body(_, carry):
        fe, ge = carry
        t = ge[:, None, :] - Ce
        m = jnp.max(t, axis=2)
        s = jnp.sum(jnp.exp2(t - m[:, :, None]), axis=2)
        fe = -(m + jnp.log2(s) + l2_b)
        u = fe[:, :, None] - Ce
        m2 = jnp.max(u, axis=1)
        s2 = jnp.sum(jnp.exp2(u - m2[:, None, :]), axis=1)
        ge = -(m2 + jnp.log2(s2) + l2_a)
        return fe, ge

    fe0 = jnp.zeros((B, S), jnp.float32)
    ge0 = jnp.zeros((B, K), jnp.float32)
    fe, ge = jax.lax.fori_loop(0, _MAX_ITER, body, (fe0, ge0))

    logP = fe[:, :, None] + ge[:, None, :] - Ce
    hist = jnp.sum(jnp.exp2(logP), axis=1)
    out_ref[...] = hist / jnp.sum(hist, axis=1, keepdims=True)


@jax.jit
def kernel(samples, codebook):
    N, S, d = samples.shape
    K = codebook.shape[0]
    B = _BLOCK_N
    grid = (N // B,)
    in_specs = [
        pl.BlockSpec((B, S, d), lambda i: (i, 0, 0)),
        pl.BlockSpec((K, d), lambda i: (0, 0)),
    ]
    params = pltpu.CompilerParams(dimension_semantics=("parallel",))

    hist, flags = pl.pallas_call(
        _fast_kernel,
        grid=grid,
        in_specs=in_specs,
        out_specs=[
            pl.BlockSpec((B, K), lambda i: (i, 0)),
            pl.BlockSpec((B, K), lambda i: (i, 0)),
        ],
        out_shape=[
            jax.ShapeDtypeStruct((N, K), jnp.float32),
            jax.ShapeDtypeStruct((N, K), jnp.float32),
        ],
        scratch_shapes=[pltpu.VMEM((B, S, K), jnp.float32)],
        compiler_params=params,
    )(samples, codebook)

    def fallback(_):
        return pl.pallas_call(
            _exact_kernel,
            grid=grid,
            in_specs=in_specs,
            out_specs=pl.BlockSpec((B, K), lambda i: (i, 0)),
            out_shape=jax.ShapeDtypeStruct((N, K), jnp.float32),
            compiler_params=params,
        )(samples, codebook)

    return jax.lax.cond(jnp.max(flags) > 0.0, fallback, lambda _: hist, 0)


# speedup vs baseline: 1.1126x; 1.1126x over previous
import functools

import jax
import jax.numpy as jnp
from jax.experimental import pallas as pl
from jax.experimental.pallas import tpu as pltpu

_EPS = 0.3 ** 2
_MAX_ITER = 100
_BLOCK_N = 8
_LOG2E = 1.4426950408889634


def _prep(samples_ref, codebook_ref):
    B, S, d = samples_ref.shape
    K = codebook_ref.shape[0]
    scale = jnp.float32(_LOG2E / _EPS)
    x = samples_ref[...]
    y = codebook_ref[...]
    x2 = jnp.sum(x * x, axis=-1)
    y2 = jnp.sum(y * y, axis=-1)
    xy = jax.lax.dot_general(
        x.reshape(B * S, d), y,
        dimension_numbers=(((1,), (1,)), ((), ())),
        preferred_element_type=jnp.float32,
    )
    D0 = ((xy - 0.5 * x2.reshape(B * S, 1)) - 0.5 * y2.reshape(1, K)) * scale
    return D0.reshape(B, S, K)


def _fast_kernel(samples_ref, codebook_ref, out_ref, flag_ref, D_ref):
    B, S, _ = samples_ref.shape
    K = codebook_ref.shape[0]
    l2_a = -jnp.log2(jnp.float32(S))
    l2_b = -jnp.log2(jnp.float32(K))
    lo = jnp.float32(2.0 ** -80)
    hi = jnp.float32(2.0 ** 80)

    D0 = _prep(samples_ref, codebook_ref)

    m = jnp.max(D0, axis=2)
    s = jnp.sum(jnp.exp2(D0 - m[:, :, None]), axis=2)
    dfe = -(m + jnp.log2(s) + l2_b)
    Dn = D0 + dfe[:, :, None]
    m2 = jnp.max(Dn, axis=1)
    s2 = jnp.sum(jnp.exp2(Dn - m2[:, None, :]), axis=1)
    dge = -(m2 + jnp.log2(s2) + l2_a)
    D_ref[...] = Dn

    def body(_, carry):
        dge, bad = carry
        Dn = D_ref[...] + dge[:, None, :]
        s = jnp.sum(jnp.exp2(Dn), axis=2)
        bad = bad | jnp.any((s <= lo) | (s >= hi) | jnp.isnan(s))
        dfe = -(jnp.log2(s) + l2_b)
        D_ref[...] = Dn + dfe[:, :, None]
        s2 = jnp.sum(jnp.exp2(D_ref[...]), axis=1)
        bad = bad | jnp.any((s2 <= lo) | (s2 >= hi) | jnp.isnan(s2))
        dge = -(jnp.log2(s2) + l2_a)
        return dge, bad

    dge, bad = jax.lax.fori_loop(
        0, _MAX_ITER - 1, body, (dge, jnp.bool_(False)), unroll=33)

    h = jnp.sum(jnp.exp2(D_ref[...] + dge[:, None, :]), axis=1)
    out_ref[...] = h / jnp.sum(h, axis=1, keepdims=True)
    flag_ref[...] = jnp.full((B, K), jnp.where(bad, 1.0, 0.0), jnp.float32)


def _exact_kernel(samples_ref, codebook_ref, out_ref):
    B, S, _ = samples_ref.shape
    K = codebook_ref.shape[0]
    l2_a = -jnp.log2(jnp.float32(S))
    l2_b = -jnp.log2(jnp.float32(K))

    Ce = -_prep(samples_ref, codebook_ref)

    def body(_, carry):
        fe, ge = carry
        t = ge[:, None, :] - Ce
        m = jnp.max(t, axis=2)
        s = jnp.sum(jnp.exp2(t - m[:, :, None]), axis=2)
        fe = -(m + jnp.log2(s) + l2_b)
        u = fe[:, :, None] - Ce
        m2 = jnp.max(u, axis=1)
        s2 = jnp.sum(jnp.exp2(u - m2[:, None, :]), axis=1)
        ge = -(m2 + jnp.log2(s2) + l2_a)
        return fe, ge

    fe0 = jnp.zeros((B, S), jnp.float32)
    ge0 = jnp.zeros((B, K), jnp.float32)
    fe, ge = jax.lax.fori_loop(0, _MAX_ITER, body, (fe0, ge0))

    logP = fe[:, :, None] + ge[:, None, :] - Ce
    hist = jnp.sum(jnp.exp2(logP), axis=1)
    out_ref[...] = hist / jnp.sum(hist, axis=1, keepdims=True)


@jax.jit
def kernel(samples, codebook):
    N, S, d = samples.shape
    K = codebook.shape[0]
    B = _BLOCK_N
    grid = (N // B,)
    in_specs = [
        pl.BlockSpec((B, S, d), lambda i: (i, 0, 0)),
        pl.BlockSpec((K, d), lambda i: (0, 0)),
    ]
    params = pltpu.CompilerParams(dimension_semantics=("parallel",))

    hist, flags = pl.pallas_call(
        _fast_kernel,
        grid=grid,
        in_specs=in_specs,
        out_specs=[
            pl.BlockSpec((B, K), lambda i: (i, 0)),
            pl.BlockSpec((B, K), lambda i: (i, 0)),
        ],
        out_shape=[
            jax.ShapeDtypeStruct((N, K), jnp.float32),
            jax.ShapeDtypeStruct((N, K), jnp.float32),
        ],
        scratch_shapes=[pltpu.VMEM((B, S, K), jnp.float32)],
        compiler_params=params,
    )(samples, codebook)

    def fallback(_):
        return pl.pallas_call(
            _exact_kernel,
            grid=grid,
            in_specs=in_specs,
            out_specs=pl.BlockSpec((B, K), lambda i: (i, 0)),
            out_shape=jax.ShapeDtypeStruct((N, K), jnp.float32),
            compiler_params=params,
        )(samples, codebook)

    return jax.lax.cond(jnp.max(flags) > 0.0, fallback, lambda _: hist, 0)


# speedup vs baseline: 1.1711x; 1.0526x over previous
import functools

import jax
import jax.numpy as jnp
from jax.experimental import pallas as pl
from jax.experimental.pallas import tpu as pltpu

_EPS = 0.3 ** 2
_MAX_ITER = 100
_BLOCK_N = 16
_LOG2E = 1.4426950408889634


def _prep(samples_ref, codebook_ref):
    B, S, d = samples_ref.shape
    K = codebook_ref.shape[0]
    scale = jnp.float32(_LOG2E / _EPS)
    x = samples_ref[...]
    y = codebook_ref[...]
    x2 = jnp.sum(x * x, axis=-1)
    y2 = jnp.sum(y * y, axis=-1)
    xy = jax.lax.dot_general(
        x.reshape(B * S, d), y,
        dimension_numbers=(((1,), (1,)), ((), ())),
        preferred_element_type=jnp.float32,
    )
    D0 = ((xy - 0.5 * x2.reshape(B * S, 1)) - 0.5 * y2.reshape(1, K)) * scale
    return D0.reshape(B, S, K)


def _fast_kernel(samples_ref, codebook_ref, out_ref, flag_ref, D_ref):
    B, S, _ = samples_ref.shape
    K = codebook_ref.shape[0]
    l2_a = -jnp.log2(jnp.float32(S))
    l2_b = -jnp.log2(jnp.float32(K))
    lo = jnp.float32(2.0 ** -80)
    hi = jnp.float32(2.0 ** 80)

    D0 = _prep(samples_ref, codebook_ref)

    m = jnp.max(D0, axis=2)
    s = jnp.sum(jnp.exp2(D0 - m[:, :, None]), axis=2)
    dfe = -(m + jnp.log2(s) + l2_b)
    Dn = D0 + dfe[:, :, None]
    m2 = jnp.max(Dn, axis=1)
    s2 = jnp.sum(jnp.exp2(Dn - m2[:, None, :]), axis=1)
    dge = -(m2 + jnp.log2(s2) + l2_a)
    D_ref[...] = Dn

    def body(_, carry):
        dge, bad = carry
        s = jnp.sum(jnp.exp2(D_ref[...] + dge[:, None, :]), axis=2)
        bad = bad | jnp.any((s <= lo) | (s >= hi) | jnp.isnan(s))
        dfe = -(jnp.log2(s) + l2_b)
        D_ref[...] = (D_ref[...] + dge[:, None, :]) + dfe[:, :, None]
        s2 = jnp.sum(jnp.exp2(D_ref[...]), axis=1)
        bad = bad | jnp.any((s2 <= lo) | (s2 >= hi) | jnp.isnan(s2))
        dge = -(jnp.log2(s2) + l2_a)
        return dge, bad

    dge, bad = jax.lax.fori_loop(
        0, _MAX_ITER - 1, body, (dge, jnp.bool_(False)), unroll=33)

    h = jnp.sum(jnp.exp2(D_ref[...] + dge[:, None, :]), axis=1)
    out_ref[...] = h / jnp.sum(h, axis=1, keepdims=True)
    flag_ref[...] = jnp.full((B, K), jnp.where(bad, 1.0, 0.0), jnp.float32)


def _exact_kernel(samples_ref, codebook_ref, out_ref):
    B, S, _ = samples_ref.shape
    K = codebook_ref.shape[0]
    l2_a = -jnp.log2(jnp.float32(S))
    l2_b = -jnp.log2(jnp.float32(K))

    Ce = -_prep(samples_ref, codebook_ref)

    def body(_, carry):
        fe, ge = carry
        t = ge[:, None, :] - Ce
        m = jnp.max(t, axis=2)
        s = jnp.sum(jnp.exp2(t - m[:, :, None]), axis=2)
        fe = -(m + jnp.log2(s) + l2_b)
        u = fe[:, :, None] - Ce
        m2 = jnp.max(u, axis=1)
        s2 = jnp.sum(jnp.exp2(u - m2[:, None, :]), axis=1)
        ge = -(m2 + jnp.log2(s2) + l2_a)
        return fe, ge

    fe0 = jnp.zeros((B, S), jnp.float32)
    ge0 = jnp.zeros((B, K), jnp.float32)
    fe, ge = jax.lax.fori_loop(0, _MAX_ITER, body, (fe0, ge0))

    logP = fe[:, :, None] + ge[:, None, :] - Ce
    hist = jnp.sum(jnp.exp2(logP), axis=1)
    out_ref[...] = hist / jnp.sum(hist, axis=1, keepdims=True)


@jax.jit
def kernel(samples, codebook):
    N, S, d = samples.shape
    K = codebook.shape[0]
    B = _BLOCK_N
    grid = (N // B,)
    in_specs = [
        pl.BlockSpec((B, S, d), lambda i: (i, 0, 0)),
        pl.BlockSpec((K, d), lambda i: (0, 0)),
    ]
    params = pltpu.CompilerParams(dimension_semantics=("parallel",))

    hist, flags = pl.pallas_call(
        _fast_kernel,
        grid=grid,
        in_specs=in_specs,
        out_specs=[
            pl.BlockSpec((B, K), lambda i: (i, 0)),
            pl.BlockSpec((B, K), lambda i: (i, 0)),
        ],
        out_shape=[
            jax.ShapeDtypeStruct((N, K), jnp.float32),
            jax.ShapeDtypeStruct((N, K), jnp.float32),
        ],
        scratch_shapes=[pltpu.VMEM((B, S, K), jnp.float32)],
        compiler_params=params,
    )(samples, codebook)

    def fallback(_):
        return pl.pallas_call(
            _exact_kernel,
            grid=grid,
            in_specs=in_specs,
            out_specs=pl.BlockSpec((B, K), lambda i: (i, 0)),
            out_shape=jax.ShapeDtypeStruct((N, K), jnp.float32),
            compiler_params=params,
        )(samples, codebook)

    return jax.lax.cond(jnp.max(flags) > 0.0, fallback, lambda _: hist, 0)
